# baseline (speedup 1.0000x reference)
import jax
import jax.numpy as jnp
from jax import lax
from jax.experimental import pallas as pl
from jax.experimental.pallas import tpu as pltpu
from jax.experimental.pallas import tpu_sc as plsc

_ROWS = 1_000_000
_D = 16
_NW = 32
_CH = 3_904
_NFULL = _ROWS // _CH
_PERW = _NFULL // _NW
_TAIL_OFF = _NFULL * _CH
_TAIL = _ROWS - _TAIL_OFF
_NSEM = 4


def _run(src, dst, *sems):
    wid = lax.axis_index("s") * 2 + lax.axis_index("c")

    def mk(g):
        cid = wid * _PERW + g
        off = pl.multiple_of(cid * _CH, 8)
        return pltpu.make_async_copy(
            src.at[pl.ds(off, _CH), :], dst.at[pl.ds(off, _CH), :], sems[g % _NSEM]
        )

    cps = [mk(g) for g in range(_PERW)]
    for g in range(_PERW):
        if g >= _NSEM:
            cps[g - _NSEM].wait()
        cps[g].start()
    for g in range(_PERW - _NSEM, _PERW):
        cps[g].wait()

    tail = pltpu.make_async_copy(
        src.at[pl.ds(_TAIL_OFF, _TAIL), :],
        dst.at[pl.ds(_TAIL_OFF, _TAIL), :],
        sems[0],
    )

    @pl.when(wid == 0)
    def _():
        tail.start()
        tail.wait()


def kernel(med_embeddings):
    run = pl.kernel(
        _run,
        out_type=jax.ShapeDtypeStruct((_ROWS, _D), jnp.float32),
        mesh=plsc.VectorSubcoreMesh(core_axis_name="c", subcore_axis_name="s"),
        scratch_types=[pltpu.SemaphoreType.DMA for _ in range(_NSEM)],
    )
    return run(med_embeddings)


# speedup vs baseline: 19.0929x; 19.0929x over previous
import jax
import jax.numpy as jnp
from jax.experimental import pallas as pl
from jax.experimental.pallas import tpu as pltpu

_ROWS = 1_000_000
_D = 16
_CH = 8_000
_NSTEPS = _ROWS // _CH
_NBUF = 6
_LAG = 3


def _copy_body(src, dst, *bufs_and_sems):
    bufs = bufs_and_sems[:_NBUF]
    sem_in, sem_out = bufs_and_sems[_NBUF], bufs_and_sems[_NBUF + 1]

    in_c = [None] * _NSTEPS
    out_c = [None] * _NSTEPS

    def issue_out(j):
        b = j % _NBUF
        in_c[j].wait()
        out_c[j] = pltpu.make_async_copy(
            bufs[b], dst.at[pl.ds(j * _CH, _CH), :], sem_out.at[b]
        )
        out_c[j].start()

    for i in range(_NSTEPS):
        b = i % _NBUF
        if i >= _NBUF:
            out_c[i - _NBUF].wait()
        in_c[i] = pltpu.make_async_copy(
            src.at[pl.ds(i * _CH, _CH), :], bufs[b], sem_in.at[b]
        )
        in_c[i].start()
        if i >= _LAG:
            issue_out(i - _LAG)
    for j in range(_NSTEPS - _LAG, _NSTEPS):
        issue_out(j)
    for j in range(_NSTEPS - _NBUF, _NSTEPS):
        out_c[j].wait()


def kernel(med_embeddings):
    return pl.pallas_call(
        _copy_body,
        in_specs=[pl.BlockSpec(memory_space=pltpu.MemorySpace.HBM)],
        out_specs=pl.BlockSpec(memory_space=pltpu.MemorySpace.HBM),
        out_shape=jax.ShapeDtypeStruct(med_embeddings.shape, med_embeddings.dtype),
        scratch_shapes=(
            [pltpu.VMEM((_CH, _D), jnp.float32) for _ in range(_NBUF)]
            + [pltpu.SemaphoreType.DMA((_NBUF,)), pltpu.SemaphoreType.DMA((_NBUF,))]
        ),
    )(med_embeddings)
